# baseline (speedup 1.0000x reference)
import functools

import jax
import jax.numpy as jnp
from jax import lax
from jax.experimental import pallas as pl
from jax.experimental.pallas import tpu as pltpu
from jax.experimental.pallas import tpu_sc as plsc

VOCAB = 1000000
D = 64
B = 1024
S = 200
F = 4

NC = 2
NS = 16
NW = NC * NS

N = B * S * F
N_W = N // NW
CHUNK = 128
NCH = N_W // CHUNK
NBUF = 4
NBLK = NCH // NBUF


def _gather_body(idx_hbm, table_hbm, out_hbm, idx_v, rows_v, gsem):
    wid = lax.axis_index("s") * NC + lax.axis_index("c")
    base = wid * N_W
    pltpu.sync_copy(idx_hbm.at[wid], idx_v)

    def blk(jb, _):
        copies = [
            pltpu.async_copy(
                table_hbm.at[idx_v.at[jb * NBUF + b]],
                rows_v.at[pl.ds(b * CHUNK, CHUNK)],
                gsem,
            )
            for b in range(NBUF)
        ]
        for cpy in copies:
            cpy.wait()
        pltpu.sync_copy(
            rows_v,
            out_hbm.at[pl.ds(base + jb * (NBUF * CHUNK), NBUF * CHUNK)],
        )
        return 0

    lax.fori_loop(0, NBLK, blk, 0)


_gather = functools.partial(
    pl.kernel,
    mesh=plsc.VectorSubcoreMesh(core_axis_name="c", subcore_axis_name="s"),
    out_type=jax.ShapeDtypeStruct((N, D), jnp.float32),
    compiler_params=pltpu.CompilerParams(use_tc_tiling_on_sc=False),
    scratch_types=[
        pltpu.VMEM((NCH, CHUNK), jnp.int32),
        pltpu.VMEM((NBUF * CHUNK, D), jnp.float32),
        pltpu.SemaphoreType.DMA,
    ],
)(_gather_body)


def kernel(table, idx_f0, idx_f1, idx_f2, idx_f3, seq_lens):
    del seq_lens
    idx = jnp.stack(
        [idx_f0.reshape(-1), idx_f1.reshape(-1), idx_f2.reshape(-1),
         idx_f3.reshape(-1)],
        axis=1,
    )
    idx = idx.reshape(NW, NCH, CHUNK)
    out = _gather(idx, table)
    return out.reshape(B, S, F * D)


# speedup vs baseline: 1.0266x; 1.0266x over previous
import functools

import jax
import jax.numpy as jnp
from jax import lax
from jax.experimental import pallas as pl
from jax.experimental.pallas import tpu as pltpu
from jax.experimental.pallas import tpu_sc as plsc

VOCAB = 1000000
D = 64
B = 1024
S = 200
F = 4

NC = 2
NS = 16
NW = NC * NS

N = B * S * F
N_W = N // NW
CHUNK = 128
NCH = N_W // CHUNK
NBUF = 4
BLKR = NBUF * CHUNK
NBLK = NCH // NBUF


def _gather_body(idx_hbm, table_hbm, out_hbm, idx_v, rows0, rows1, gsem, osem):
    wid = lax.axis_index("s") * NC + lax.axis_index("c")
    base = wid * N_W
    pltpu.sync_copy(idx_hbm.at[wid], idx_v)
    bufs = (rows0, rows1)

    def fill(jb, buf):
        copies = [
            pltpu.async_copy(
                table_hbm.at[idx_v.at[jb * NBUF + b]],
                buf.at[pl.ds(b * CHUNK, CHUNK)],
                gsem,
            )
            for b in range(NBUF)
        ]
        for cpy in copies:
            cpy.wait()
        pltpu.async_copy(buf, out_hbm.at[pl.ds(base + jb * BLKR, BLKR)], osem)

    for p in range(2):
        fill(p, bufs[p])

    def blk2(j2, _):
        for p in range(2):
            pltpu.make_async_copy(
                bufs[p], out_hbm.at[pl.ds(base, BLKR)], osem
            ).wait()
            fill(j2 * 2 + p, bufs[p])
        return 0

    lax.fori_loop(1, NBLK // 2, blk2, 0)

    for p in range(2):
        pltpu.make_async_copy(
            bufs[p], out_hbm.at[pl.ds(base, BLKR)], osem
        ).wait()


_gather = functools.partial(
    pl.kernel,
    mesh=plsc.VectorSubcoreMesh(core_axis_name="c", subcore_axis_name="s"),
    out_type=jax.ShapeDtypeStruct((N, D), jnp.float32),
    compiler_params=pltpu.CompilerParams(use_tc_tiling_on_sc=False),
    scratch_types=[
        pltpu.VMEM((NCH, CHUNK), jnp.int32),
        pltpu.VMEM((BLKR, D), jnp.float32),
        pltpu.VMEM((BLKR, D), jnp.float32),
        pltpu.SemaphoreType.DMA,
        pltpu.SemaphoreType.DMA,
    ],
)(_gather_body)


def kernel(table, idx_f0, idx_f1, idx_f2, idx_f3, seq_lens):
    del seq_lens
    idx = jnp.stack(
        [idx_f0.reshape(-1), idx_f1.reshape(-1), idx_f2.reshape(-1),
         idx_f3.reshape(-1)],
        axis=1,
    )
    idx = idx.reshape(NW, NCH, CHUNK)
    out = _gather(idx, table)
    return out.reshape(B, S, F * D)


# speedup vs baseline: 1.1376x; 1.1082x over previous
import functools

import jax
import jax.numpy as jnp
from jax import lax
from jax.experimental import pallas as pl
from jax.experimental.pallas import tpu as pltpu
from jax.experimental.pallas import tpu_sc as plsc

VOCAB = 1000000
D = 64
B = 1024
S = 200
F = 4

NC = 2
NS = 16
NW = NC * NS

N = B * S * F
N_W = N // NW
CHUNK = 128
NCH = N_W // CHUNK
NBUF = 2
BLKR = NBUF * CHUNK
NBLK = NCH // NBUF
K = 4


def _gather_body(
    idx_hbm, table_hbm, out_hbm, idx_v,
    rows0, rows1, rows2, rows3,
    g0, g1, g2, g3, o0, o1, o2, o3,
):
    wid = lax.axis_index("s") * NC + lax.axis_index("c")
    base = wid * N_W
    pltpu.sync_copy(idx_hbm.at[wid], idx_v)
    bufs = (rows0, rows1, rows2, rows3)
    gsems = (g0, g1, g2, g3)
    osems = (o0, o1, o2, o3)

    def issue(jb, p):
        for b in range(NBUF):
            pltpu.async_copy(
                table_hbm.at[idx_v.at[jb * NBUF + b]],
                bufs[p].at[pl.ds(b * CHUNK, CHUNK)],
                gsems[p],
            )

    def wait_gathers(p):
        for b in range(NBUF):
            pltpu.make_async_copy(
                table_hbm.at[idx_v.at[0]],
                bufs[p].at[pl.ds(b * CHUNK, CHUNK)],
                gsems[p],
            ).wait()

    def writeback(jb, p):
        pltpu.async_copy(
            bufs[p], out_hbm.at[pl.ds(base + jb * BLKR, BLKR)], osems[p]
        )

    def wait_writeback(p):
        pltpu.make_async_copy(
            bufs[p], out_hbm.at[pl.ds(base, BLKR)], osems[p]
        ).wait()

    for j in range(K - 1):
        issue(j, j)

    for j in range(K):
        p = j % K
        wait_gathers(p)
        writeback(j, p)
        nxt = j + K - 1
        np_ = nxt % K
        if nxt >= K:
            wait_writeback(np_)
        issue(nxt, np_)

    def group(go, _):
        for p in range(K):
            j = go * K + p
            wait_gathers(p)
            writeback(j, p)
            np_ = (p + K - 1) % K
            wait_writeback(np_)
            issue(j + K - 1, np_)
        return 0

    lax.fori_loop(1, NBLK // K - 1, group, 0)

    for j in range(NBLK - K, NBLK):
        p = j % K
        wait_gathers(p)
        writeback(j, p)
        nxt = j + K - 1
        if nxt < NBLK:
            np_ = nxt % K
            wait_writeback(np_)
            issue(nxt, np_)

    for p in range(K):
        wait_writeback(p)


_gather = functools.partial(
    pl.kernel,
    mesh=plsc.VectorSubcoreMesh(core_axis_name="c", subcore_axis_name="s"),
    out_type=jax.ShapeDtypeStruct((N, D), jnp.float32),
    compiler_params=pltpu.CompilerParams(use_tc_tiling_on_sc=False),
    scratch_types=[
        pltpu.VMEM((NCH, CHUNK), jnp.int32),
        pltpu.VMEM((BLKR, D), jnp.float32),
        pltpu.VMEM((BLKR, D), jnp.float32),
        pltpu.VMEM((BLKR, D), jnp.float32),
        pltpu.VMEM((BLKR, D), jnp.float32),
        pltpu.SemaphoreType.DMA,
        pltpu.SemaphoreType.DMA,
        pltpu.SemaphoreType.DMA,
        pltpu.SemaphoreType.DMA,
        pltpu.SemaphoreType.DMA,
        pltpu.SemaphoreType.DMA,
        pltpu.SemaphoreType.DMA,
        pltpu.SemaphoreType.DMA,
    ],
)(_gather_body)


def kernel(table, idx_f0, idx_f1, idx_f2, idx_f3, seq_lens):
    del seq_lens
    idx = jnp.stack(
        [idx_f0, idx_f1, idx_f2, idx_f3], axis=-1
    )
    idx = idx.reshape(B, S // 8, 8, 2, 2)
    idx = idx.transpose(0, 1, 3, 2, 4)
    idx = idx.reshape(NW, NCH, CHUNK)
    out = _gather(idx, table)
    out = out.reshape(B, S // 8, 2, 8, 2 * D)
    out = out.transpose(0, 1, 3, 2, 4)
    return out.reshape(B, S, F * D)
